# Initial kernel scaffold; baseline (speedup 1.0000x reference)
import jax
import jax.numpy as jnp
from jax.experimental import pallas as pl


def kernel(x, y, A):
    raise NotImplementedError("write your pallas kernel here")


# speedup vs baseline: 29.6005x; 29.6005x over previous
import functools

import jax
import jax.numpy as jnp
from jax.experimental import pallas as pl
from jax.experimental.pallas import tpu as pltpu

L = 17
B = 128
S = 512


def _crf_kernel(xt_ref, yt_ref, A_ref, AT_ref, path_ref, nll_ref, bp_ref):
    A = A_ref[...]
    AT = AT_ref[...]
    EAT = jnp.exp(AT)

    lane_iota = jax.lax.broadcasted_iota(jnp.int32, (L, B), 0)

    x0 = xt_ref[0]
    y0 = yt_ref[pl.ds(0, 1), :]

    dp0 = x0
    m0 = jnp.max(x0, axis=0, keepdims=True)
    p0 = jnp.exp(x0 - m0)
    alpha0 = m0 + jnp.log(
        jax.lax.dot(EAT, p0, preferred_element_type=jnp.float32))
    emit0 = jnp.where(lane_iota == y0, x0, 0.0)
    trans0 = jnp.zeros((L, B), jnp.float32)

    def step(j, carry):
        dp, alpha, emitacc, transacc, yprev = carry
        xj = xt_ref[j]
        yj = yt_ref[pl.ds(j, 1), :]

        best = (dp[0:1, :] + AT[:, 0:1]) + xj
        besti = jnp.zeros((L, B), jnp.int32)
        for k in range(1, L):
            cand = (dp[k:k + 1, :] + AT[:, k:k + 1]) + xj
            gt = cand > best
            best = jnp.where(gt, cand, best)
            besti = jnp.where(gt, k, besti)
        bp_ref[j] = besti

        u = xj + alpha
        m = jnp.max(u, axis=0, keepdims=True)
        p = jnp.exp(u - m)
        alpha_new = m + jnp.log(
            jax.lax.dot(EAT, p, preferred_element_type=jnp.float32))

        ohprev = (lane_iota == yprev).astype(jnp.float32)
        acols = jax.lax.dot(A, ohprev,
                            preferred_element_type=jnp.float32)
        emitacc = emitacc + jnp.where(lane_iota == yj, xj, 0.0)
        transacc = transacc + jnp.where(lane_iota == yj, acols, 0.0)
        return best, alpha_new, emitacc, transacc, yj

    dp, alpha, emitacc, transacc, yprev = jax.lax.fori_loop(
        1, S - 1, step, (dp0, alpha0, emit0, trans0, y0))

    xl = xt_ref[S - 1]
    yl = yt_ref[pl.ds(S - 1, 1), :]
    best = (dp[0:1, :] + AT[:, 0:1]) + xl
    besti = jnp.zeros((L, B), jnp.int32)
    for k in range(1, L):
        cand = (dp[k:k + 1, :] + AT[:, k:k + 1]) + xl
        gt = cand > best
        best = jnp.where(gt, cand, best)
        besti = jnp.where(gt, k, besti)
    bp_ref[S - 1] = besti
    dp_last = best

    v = xl + alpha
    mz = jnp.max(v, axis=0, keepdims=True)
    z = mz + jnp.log(jnp.sum(jnp.exp(v - mz), axis=0, keepdims=True))

    ohprev = (lane_iota == yprev).astype(jnp.float32)
    acols = jax.lax.dot(A, ohprev, preferred_element_type=jnp.float32)
    emitacc = emitacc + jnp.where(lane_iota == yl, xl, 0.0)
    transacc = transacc + jnp.where(lane_iota == yl, acols, 0.0)

    s = jnp.sum(emitacc + transacc, axis=0, keepdims=True)
    nll_ref[...] = jnp.sum(z - s, axis=1, keepdims=True) * (1.0 / B)

    last = jnp.zeros((1, B), jnp.int32)
    bestv = dp_last[0:1, :]
    for k in range(1, L):
        row = dp_last[k:k + 1, :]
        gt = row > bestv
        bestv = jnp.where(gt, row, bestv)
        last = jnp.where(gt, k, last)
    path_ref[pl.ds(S - 1, 1), :] = last

    def back(t, cur):
        j = S - 1 - t
        bprow = bp_ref[j]
        prev = jnp.max(jnp.where(lane_iota == cur, bprow, 0),
                       axis=0, keepdims=True)
        path_ref[pl.ds(j - 1, 1), :] = prev
        return prev

    jax.lax.fori_loop(0, S - 1, back, last)


@functools.partial(jax.jit, static_argnames=())
def kernel(x, y, A):
    xt = jnp.transpose(x, (1, 2, 0))
    yt = jnp.transpose(y, (1, 0))
    AT = jnp.transpose(A, (1, 0))

    path_t, nll = pl.pallas_call(
        _crf_kernel,
        out_shape=(
            jax.ShapeDtypeStruct((S, B), jnp.int32),
            jax.ShapeDtypeStruct((1, 1), jnp.float32),
        ),
        scratch_shapes=[pltpu.VMEM((S, L, B), jnp.int32)],
    )(xt, yt, A, AT)

    return path_t.T, nll[0, 0]
